# baseline (speedup 1.0000x reference)
import jax
import jax.numpy as jnp
from jax.experimental import pallas as pl
from jax.experimental.pallas import tpu as pltpu

N = 10000
E = 320000
DIM = 128
G = 64
K = 30

NPAD = 10240
B = 2560
NEB = E // B
NC = 512
NCH = NPAD // NC


def _deg_kernel(dst_ref, dinv_ref):
    eb = pl.program_id(0)

    @pl.when(eb == 0)
    def _():
        dinv_ref[...] = jnp.ones_like(dinv_ref)

    d = dst_ref[0]
    for nc in range(NCH):
        iota = jax.lax.broadcasted_iota(jnp.int32, (NC, B), 0) + nc * NC
        onehot = (d == iota).astype(jnp.float32)
        dinv_ref[nc * NC:(nc + 1) * NC, :] += jnp.sum(onehot, axis=1,
                                                      keepdims=True)

    @pl.when(eb == NEB - 1)
    def _():
        dinv_ref[...] = jax.lax.rsqrt(dinv_ref[...])


def _gcn_kernel(src_ref, dst_ref, h_ref, w_ref, b_ref, dinv_ref, out_ref,
                y_ref):
    eb = pl.program_id(0)

    @pl.when(eb == 0)
    def _():
        xw = jnp.dot(h_ref[...], w_ref[...],
                     preferred_element_type=jnp.float32)
        y_ref[...] = dinv_ref[...] * xw
        out_ref[...] = jnp.zeros_like(out_ref)

    s = src_ref[0]
    d = dst_ref[0]

    g = jnp.zeros((B, 32), jnp.float32)
    for nc in range(NCH):
        iota = jax.lax.broadcasted_iota(jnp.int32, (NC, B), 0) + nc * NC
        oh_s = (s == iota).astype(jnp.float32)
        g = g + jax.lax.dot_general(
            oh_s, y_ref[nc * NC:(nc + 1) * NC, :], (((0,), (0,)), ((), ())),
            preferred_element_type=jnp.float32)

    for nc in range(NCH):
        iota = jax.lax.broadcasted_iota(jnp.int32, (NC, B), 0) + nc * NC
        oh_d = (d == iota).astype(jnp.float32)
        out_ref[nc * NC:(nc + 1) * NC, :] += jnp.dot(
            oh_d, g, preferred_element_type=jnp.float32)

    @pl.when(eb == NEB - 1)
    def _():
        out_ref[...] = jnp.tanh(
            dinv_ref[...] * (out_ref[...] + y_ref[...]) + b_ref[...])


def _gcn_layer(src3, dst3, h, w, b, dinv):
    cin = h.shape[1]
    return pl.pallas_call(
        _gcn_kernel,
        grid=(NEB,),
        in_specs=[
            pl.BlockSpec((1, 1, B), lambda eb: (eb, 0, 0)),
            pl.BlockSpec((1, 1, B), lambda eb: (eb, 0, 0)),
            pl.BlockSpec((NPAD, cin), lambda eb: (0, 0)),
            pl.BlockSpec((cin, 32), lambda eb: (0, 0)),
            pl.BlockSpec((1, 32), lambda eb: (0, 0)),
            pl.BlockSpec((NPAD, 1), lambda eb: (0, 0)),
        ],
        out_specs=pl.BlockSpec((NPAD, 32), lambda eb: (0, 0)),
        out_shape=jax.ShapeDtypeStruct((NPAD, 32), jnp.float32),
        scratch_shapes=[pltpu.VMEM((NPAD, 32), jnp.float32)],
    )(src3, dst3, h, w, b, dinv)


def kernel(x, edge_index, batch, w1, b1, w2, b2, w3, b3, w4, b4, c1w, c1b,
           c2w, c2b, l1w, l1b, l2w, l2b):
    src3 = edge_index[0].reshape(NEB, 1, B)
    dst3 = edge_index[1].reshape(NEB, 1, B)

    dinv = pl.pallas_call(
        _deg_kernel,
        grid=(NEB,),
        in_specs=[pl.BlockSpec((1, 1, B), lambda eb: (eb, 0, 0))],
        out_specs=pl.BlockSpec((NPAD, 1), lambda eb: (0, 0)),
        out_shape=jax.ShapeDtypeStruct((NPAD, 1), jnp.float32),
    )(dst3)

    h0 = jnp.zeros((NPAD, DIM), jnp.float32).at[:N].set(x)
    w4p = jnp.zeros((32, 32), jnp.float32).at[:, :1].set(w4)
    b4p = jnp.zeros((32,), jnp.float32).at[:1].set(b4)

    h1 = _gcn_layer(src3, dst3, h0, w1, b1.reshape(1, 32), dinv)
    h2 = _gcn_layer(src3, dst3, h1, w2, b2.reshape(1, 32), dinv)
    h3 = _gcn_layer(src3, dst3, h2, w3, b3.reshape(1, 32), dinv)
    h4 = _gcn_layer(src3, dst3, h3, w4p, b4p.reshape(1, 32), dinv)

    h = jnp.concatenate([h1[:N], h2[:N], h3[:N], h4[:N, :1]], axis=-1)

    last = h[:, -1]
    order = jnp.lexsort((-last, batch))
    h_s = h[order]
    counts = jnp.bincount(batch, length=G)
    starts = jnp.cumsum(counts) - counts
    pos = starts[:, None] + jnp.arange(K)[None, :]
    mask = (jnp.arange(K)[None, :] < counts[:, None]).astype(h.dtype)
    gathered = h_s[jnp.clip(pos, 0, h.shape[0] - 1)] * mask[:, :, None]
    hp = gathered.reshape(G, K * h.shape[1])

    hp = hp[:, None, :]
    dn = ('NCH', 'OIH', 'NCH')
    hp = jax.lax.conv_general_dilated(hp, c1w, (97,), 'VALID',
                                      dimension_numbers=dn)
    hp = jax.nn.relu(hp + c1b[None, :, None])
    hp = jax.lax.reduce_window(hp, -jnp.inf, jax.lax.max, (1, 1, 2),
                               (1, 1, 2), 'VALID')
    hp = jax.lax.conv_general_dilated(hp, c2w, (1,), 'VALID',
                                      dimension_numbers=dn)
    hp = jax.nn.relu(hp + c2b[None, :, None])
    hp = hp.reshape(hp.shape[0], -1)
    hp = jax.nn.relu(hp @ l1w.T + l1b)
    return jax.nn.sigmoid(hp @ l2w.T + l2b)
